# baseline (speedup 1.0000x reference)
import jax
import jax.numpy as jnp
from jax.experimental import pallas as pl
from jax.experimental.pallas import tpu as pltpu

_BB = 10


def _body(nodes_ref, pooled_ref, mask_ref, locked_ref, out_ref):
    lock = mask_ref[0] & locked_ref[0]
    keep = (1.0 - lock.astype(jnp.float32))[:, :, None]
    out_ref[...] = nodes_ref[...] + pooled_ref[...] * keep


def kernel(nodes, mask, pooled_edges, locked_nodes):
    B, N, D = nodes.shape
    nb = B // _BB
    mask3 = mask.reshape(nb, _BB, N)
    locked3 = locked_nodes.reshape(nb, _BB, N)
    bs3 = pl.BlockSpec((_BB, N, D), lambda i: (i, 0, 0))
    bsm = pl.BlockSpec((1, _BB, N), lambda i: (i, 0, 0))
    return pl.pallas_call(
        _body,
        grid=(nb,),
        in_specs=[bs3, bs3, bsm, bsm],
        out_specs=bs3,
        out_shape=jax.ShapeDtypeStruct((B, N, D), nodes.dtype),
        compiler_params=pltpu.CompilerParams(
            dimension_semantics=("parallel",),
        ),
    )(nodes, pooled_edges, mask3, locked3)


# speedup vs baseline: 1.0542x; 1.0542x over previous
import jax
import jax.numpy as jnp
from jax.experimental import pallas as pl
from jax.experimental.pallas import tpu as pltpu

_BB = 25


def _body(nodes_ref, pooled_ref, mask_ref, locked_ref, out_ref):
    lock = mask_ref[0] & locked_ref[0]
    keep = (1.0 - lock.astype(jnp.float32))[:, :, None]
    out_ref[...] = nodes_ref[...] + pooled_ref[...] * keep


def kernel(nodes, mask, pooled_edges, locked_nodes):
    B, N, D = nodes.shape
    nb = B // _BB
    mask3 = mask.reshape(nb, _BB, N)
    locked3 = locked_nodes.reshape(nb, _BB, N)
    bs3 = pl.BlockSpec((_BB, N, D), lambda i: (i, 0, 0))
    bsm = pl.BlockSpec((1, _BB, N), lambda i: (i, 0, 0))
    return pl.pallas_call(
        _body,
        grid=(nb,),
        in_specs=[bs3, bs3, bsm, bsm],
        out_specs=bs3,
        out_shape=jax.ShapeDtypeStruct((B, N, D), nodes.dtype),
        compiler_params=pltpu.CompilerParams(
            dimension_semantics=("parallel",),
        ),
    )(nodes, pooled_edges, mask3, locked3)


# speedup vs baseline: 1.0666x; 1.0117x over previous
import jax
import jax.numpy as jnp
from jax.experimental import pallas as pl
from jax.experimental.pallas import tpu as pltpu

_BB = 32


def _body(nodes_ref, pooled_ref, maskf_ref, lockedf_ref, out_ref):
    lock = maskf_ref[...] * lockedf_ref[...]
    keep = (1.0 - lock)[:, :, None]
    out_ref[...] = nodes_ref[...] + pooled_ref[...] * keep


def kernel(nodes, mask, pooled_edges, locked_nodes):
    B, N, D = nodes.shape
    maskf = mask.astype(jnp.float32)
    lockedf = locked_nodes.astype(jnp.float32)
    bs3 = pl.BlockSpec((_BB, N, D), lambda i: (i, 0, 0))
    bsm = pl.BlockSpec((_BB, N), lambda i: (i, 0))
    return pl.pallas_call(
        _body,
        grid=(pl.cdiv(B, _BB),),
        in_specs=[bs3, bs3, bsm, bsm],
        out_specs=bs3,
        out_shape=jax.ShapeDtypeStruct((B, N, D), nodes.dtype),
        compiler_params=pltpu.CompilerParams(
            dimension_semantics=("parallel",),
        ),
    )(nodes, pooled_edges, maskf, lockedf)
